# baseline (speedup 1.0000x reference)
import functools

import jax
import jax.numpy as jnp
from jax.experimental import pallas as pl
from jax.experimental.pallas import tpu as pltpu

f32 = jnp.float32
bf16 = jnp.bfloat16

_DN = ('NCHW', 'OIHW', 'NCHW')
_NBINS = 256
_NPIX = 256
_NPATCH = 256


def _conv_xla(x, w, b, stride=1, pad=1):
    y = jax.lax.conv_general_dilated(x, w, (stride, stride), [(pad, pad), (pad, pad)],
                                     dimension_numbers=_DN)
    return y + b[None, :, None, None]


def _ent_kernel(v_ref, ent_ref, pdf_scr):
    v = v_ref[0]

    def body(k, carry):
        rows = []
        for d in range(8):
            m = k * 8 + d
            b = m.astype(f32) * (1.0 / 255.0)
            t = (v - b) * 100.0
            w = jnp.exp(-0.5 * (t * t))
            rows.append(jnp.sum(w, axis=0, keepdims=True))
        pdf_scr[pl.ds(k, 1)] = jnp.concatenate(rows, axis=0)[None]
        return carry

    jax.lax.fori_loop(0, _NBINS // 8, body, 0)
    pdf = pdf_scr[...].reshape(_NBINS, _NPATCH)
    s = jnp.sum(pdf, axis=0, keepdims=True)
    pn = jnp.maximum(pdf * (1.0 / s), 1e-10)
    ent_ref[...] = -jnp.sum(pn * jnp.log2(pn), axis=0, keepdims=True)[None]


def _entropy_map(gray):
    bsz = gray.shape[0]
    p = gray.reshape(bsz, 16, 16, 16, 16).transpose(0, 2, 4, 1, 3)
    p = p.reshape(bsz, _NPIX, _NPATCH)
    ent = pl.pallas_call(
        _ent_kernel,
        grid=(bsz,),
        in_specs=[pl.BlockSpec((1, _NPIX, _NPATCH), lambda b: (b, 0, 0))],
        out_specs=pl.BlockSpec((1, 1, _NPATCH), lambda b: (b, 0, 0)),
        out_shape=jax.ShapeDtypeStruct((bsz, 1, _NPATCH), jnp.float32),
        scratch_shapes=[pltpu.VMEM((_NBINS // 8, 8, _NPATCH), jnp.float32)],
        compiler_params=pltpu.CompilerParams(dimension_semantics=("parallel",)),
    )(p)
    return ent.reshape(bsz, 16, 16)


def _split(a):
    hi = a.astype(bf16)
    lo = (a - hi.astype(f32)).astype(bf16)
    return hi, lo


def _stage_body(shifts, ch, nchunks, wp, nout, act, split_out,
                ih_ref, il_ref, wh_ref, wl_ref, b_ref, *outs):
    m0 = wp + 8
    exn = ch + 2 * wp + 16

    def chunk(c, carry):
        base = pl.multiple_of(c * ch, 8)
        eh = ih_ref[0, pl.ds(base, exn), :]
        el = il_ref[0, pl.ds(base, exn), :]
        acc = jnp.zeros((ch, nout), f32)
        for i, s in enumerate(shifts):
            off = m0 + s
            lh = eh[off:off + ch]
            ll = el[off:off + ch]
            wh = wh_ref[i]
            wl = wl_ref[i]
            acc = acc + jnp.dot(lh, wh, preferred_element_type=f32)
            acc = acc + jnp.dot(lh, wl, preferred_element_type=f32)
            acc = acc + jnp.dot(ll, wh, preferred_element_type=f32)
        acc = acc + b_ref[...]
        if act == 'relu':
            acc = jnp.maximum(acc, 0.0)
        elif act == 'tanh':
            acc = jnp.tanh(acc)
        if split_out:
            yh = acc.astype(bf16)
            outs[0][0, pl.ds(base, ch), :] = yh
            outs[1][0, pl.ds(base, ch), :] = (acc - yh.astype(f32)).astype(bf16)
        else:
            outs[0][0, pl.ds(base, ch), :] = acc
        return carry

    jax.lax.fori_loop(0, nchunks, chunk, 0)


def _stage(xpair, wpair, b, shifts, wp, mout, ch, act='none', split_out=True):
    xh, xl = xpair
    wh, wl = wpair
    g, minr, k = xh.shape
    t, _, n = wh.shape
    body = functools.partial(_stage_body, tuple(shifts), ch, mout // ch, wp, n,
                             act, split_out)
    n_out = 2 if split_out else 1
    odt = bf16 if split_out else f32
    outs = pl.pallas_call(
        body,
        grid=(g,),
        in_specs=[
            pl.BlockSpec((1, minr, k), lambda i: (i, 0, 0)),
            pl.BlockSpec((1, minr, k), lambda i: (i, 0, 0)),
            pl.BlockSpec((t, k, n), lambda i: (0, 0, 0)),
            pl.BlockSpec((t, k, n), lambda i: (0, 0, 0)),
            pl.BlockSpec((1, n), lambda i: (0, 0)),
        ],
        out_specs=[pl.BlockSpec((1, mout, n), lambda i: (i, 0, 0))] * n_out,
        out_shape=[jax.ShapeDtypeStruct((g, mout, n), odt)] * n_out,
        compiler_params=pltpu.CompilerParams(dimension_semantics=("parallel",)),
    )(xh, xl, wh, wl, b)
    return outs


def _round8(n):
    return (n + 7) // 8 * 8


def _build_in(pair, h, w):
    wp = _round8(w + 2)
    m0 = wp + 8
    out = []
    for a in pair:
        g, _, _, c = a.shape
        ap = jnp.pad(a, ((0, 0), (1, 1), (1, wp - w - 1), (0, 0)))
        flat = ap.reshape(g, (h + 2) * wp, c)
        out.append(jnp.pad(flat, ((0, 0), (m0, wp + 8), (0, 0))))
    return tuple(out), wp


def _interior(pair, h, w):
    wp = _round8(w + 2)
    return tuple(a.reshape(a.shape[0], h + 2, wp, a.shape[-1])[:, 1:h + 1, 1:w + 1, :]
                 for a in pair)


def _merge_parities(planes):
    out = []
    for j in range(2):
        p00, p01, p10, p11 = (pl4[j] for pl4 in planes)
        g, h, w, c = p00.shape
        q0 = jnp.stack([p00, p01], 3).reshape(g, h, 2 * w, c)
        q1 = jnp.stack([p10, p11], 3).reshape(g, h, 2 * w, c)
        out.append(jnp.stack([q0, q1], 2).reshape(g, 2 * h, 2 * w, c))
    return tuple(out)


def _conv_taps(w9):
    return w9.transpose(2, 3, 1, 0).reshape(9, w9.shape[1], w9.shape[0])


def _conv_shifts(wp):
    return [(di - 1) * wp + (dj - 1) for di in range(3) for dj in range(3)]


def _deconv_taps(w):
    wt = jnp.flip(w, (2, 3)).transpose(1, 0, 2, 3)
    return wt.transpose(2, 3, 1, 0)


def _deconv_plane_specs(wtap, wp):
    return [
        ([0], jnp.stack([wtap[1, 1]])),
        ([0, 1], jnp.stack([wtap[1, 0], wtap[1, 2]])),
        ([0, wp], jnp.stack([wtap[0, 1], wtap[2, 1]])),
        ([0, 1, wp, wp + 1],
         jnp.stack([wtap[0, 0], wtap[0, 2], wtap[2, 0], wtap[2, 2]])),
    ]


def _deconv_stage(xint_pair, w, b, h, w_sp, ch):
    xflat, wp = _build_in(xint_pair, h, w_sp)
    wtap = _deconv_taps(w)
    bb = b[None, :]
    mout = (h + 2) * wp
    planes = []
    for shifts, taps in _deconv_plane_specs(wtap, wp):
        tpair = _split(taps)
        o = _stage(xflat, tpair, bb, shifts, wp, mout, ch, act='relu')
        planes.append(_interior(o, h, w_sp))
    return _merge_parities(planes)


def _conv_stage(xint_pair, w, b, h, w_sp, ch, act, split_out=True):
    xflat, wp = _build_in(xint_pair, h, w_sp)
    taps = _conv_taps(w)
    tpair = _split(taps)
    mout = (h + 2) * wp
    o = _stage(xflat, tpair, b[None, :], _conv_shifts(wp), wp, mout, ch,
               act=act, split_out=split_out)
    return _interior(o, h, w_sp)


def _strips(pair, h, w, n_strips):
    wp = _round8(w + 2)
    hs = h // n_strips
    out = []
    for a in pair:
        b, _, _, c = a.shape
        ap = jnp.pad(a, ((0, 0), (1, 1), (1, wp - w - 1), (0, 0)))
        st = jnp.stack([ap[:, hs * s:hs * s + hs + 2] for s in range(n_strips)], 1)
        st = st.reshape(b * n_strips, (hs + 2) * wp, c)
        out.append(jnp.pad(st, ((0, 0), (wp + 8, wp + 8), (0, 0))))
    return tuple(out), wp


def _unstrips(pair, h, w, n_strips, bsz):
    wp = _round8(w + 2)
    hs = h // n_strips
    out = []
    for a in pair:
        c = a.shape[-1]
        v = a.reshape(bsz, n_strips, hs + 2, wp, c)[:, :, 1:hs + 1]
        v = v.reshape(bsz, h, wp, c)[:, :, 1:w + 1]
        out.append(v)
    return tuple(out)


def _conv_stage_strips(xint_pair, w, b, h, w_sp, n_strips, ch, act,
                       split_out=True):
    xflat, wp = _strips(xint_pair, h, w_sp, n_strips)
    taps = _conv_taps(w)
    tpair = _split(taps)
    hs = h // n_strips
    mout = (hs + 2) * wp
    o = _stage(xflat, tpair, b[None, :], _conv_shifts(wp), wp, mout, ch,
               act=act, split_out=split_out)
    return _unstrips(o, h, w_sp, n_strips, xint_pair[0].shape[0])


def kernel(x, We0, be0, We1, be1, Wi, bi, Wd1, bd1, Wc1, bc1, Wd2, bd2,
           Wc2, bc2, Wd3, bd3, Wc3, bc3, Wo, bo):
    bsz = x.shape[0]
    lat_fine = _conv_xla(x, We0, be0, stride=8)
    lat_coarse = _conv_xla(x, We1, be1, stride=16)
    gray = 0.299 * x[:, 0] + 0.587 * x[:, 1] + 0.114 * x[:, 2]
    ent = _entropy_map(gray)
    thr = jnp.quantile(ent.reshape(-1), 0.5)
    grain = (ent > thr).astype(x.dtype)
    coarse_up = jnp.repeat(jnp.repeat(lat_coarse, 2, axis=2), 2, axis=3)
    g = jnp.repeat(jnp.repeat(grain, 2, axis=1), 2, axis=2)[:, None]
    routed = g * lat_fine + (1.0 - g) * coarse_up

    r = routed.transpose(0, 2, 3, 1)
    r = jnp.pad(r, ((0, 0), (0, 0), (0, 0), (0, 4)))
    hcur = _conv_stage(_split(r), jnp.pad(Wi, ((0, 0), (0, 4), (0, 0), (0, 0))),
                       bi, 32, 32, ch=680, act='none')
    hcur = _deconv_stage(hcur, Wd1, bd1, 32, 32, ch=680)
    hcur = _conv_stage(hcur, Wc1, bc1, 64, 64, ch=792, act='relu')
    hcur = _deconv_stage(hcur, Wd2, bd2, 64, 64, ch=792)
    hcur = _conv_stage(hcur, Wc2, bc2, 128, 128, ch=1768, act='relu')
    hcur = _deconv_stage(hcur, Wd3, bd3, 128, 128, ch=1768)
    hcur = _conv_stage_strips(hcur, Wc3, bc3, 256, 256, n_strips=4, ch=1584,
                              act='relu')
    wo8 = jnp.pad(Wo, ((0, 5), (0, 0), (0, 0), (0, 0)))
    bo8 = jnp.pad(bo, ((0, 5),))
    rec8 = _conv_stage_strips(hcur, wo8, bo8, 256, 256, n_strips=4, ch=1584,
                              act='tanh', split_out=False)
    rec = rec8[0][..., :3].transpose(0, 3, 1, 2)
    return rec, routed, grain, ent


# speedup vs baseline: 1.8622x; 1.8622x over previous
import functools

import jax
import jax.numpy as jnp
from jax.experimental import pallas as pl
from jax.experimental.pallas import tpu as pltpu

f32 = jnp.float32
bf16 = jnp.bfloat16

_DN = ('NCHW', 'OIHW', 'NCHW')
_NBINS = 256
_NPIX = 256
_NPATCH = 256


def _conv_xla(x, w, b, stride=1, pad=1):
    y = jax.lax.conv_general_dilated(x, w, (stride, stride), [(pad, pad), (pad, pad)],
                                     dimension_numbers=_DN)
    return y + b[None, :, None, None]


def _ent_kernel(v_ref, ent_ref, pdf_scr):
    v = v_ref[0]

    def body(k, carry):
        rows = []
        for d in range(8):
            m = k * 8 + d
            b = m.astype(f32) * (1.0 / 255.0)
            t = (v - b) * 100.0
            w = jnp.exp(-0.5 * (t * t))
            rows.append(jnp.sum(w, axis=0, keepdims=True))
        pdf_scr[pl.ds(k, 1)] = jnp.concatenate(rows, axis=0)[None]
        return carry

    jax.lax.fori_loop(0, _NBINS // 8, body, 0)
    pdf = pdf_scr[...].reshape(_NBINS, _NPATCH)
    s = jnp.sum(pdf, axis=0, keepdims=True)
    pn = jnp.maximum(pdf * (1.0 / s), 1e-10)
    ent_ref[...] = -jnp.sum(pn * jnp.log2(pn), axis=0, keepdims=True)[None]


def _entropy_map(gray):
    bsz = gray.shape[0]
    p = gray.reshape(bsz, 16, 16, 16, 16).transpose(0, 2, 4, 1, 3)
    p = p.reshape(bsz, _NPIX, _NPATCH)
    ent = pl.pallas_call(
        _ent_kernel,
        grid=(bsz,),
        in_specs=[pl.BlockSpec((1, _NPIX, _NPATCH), lambda b: (b, 0, 0))],
        out_specs=pl.BlockSpec((1, 1, _NPATCH), lambda b: (b, 0, 0)),
        out_shape=jax.ShapeDtypeStruct((bsz, 1, _NPATCH), jnp.float32),
        scratch_shapes=[pltpu.VMEM((_NBINS // 8, 8, _NPATCH), jnp.float32)],
        compiler_params=pltpu.CompilerParams(dimension_semantics=("parallel",)),
    )(p)
    return ent.reshape(bsz, 16, 16)


def _split(a):
    hi = a.astype(bf16)
    lo = (a - hi.astype(f32)).astype(bf16)
    return hi, lo


def _stage_body(groups, ch, nchunks, wp, nout, npack, act, mask, strip_h,
                strip_grid, x_ref, b_ref, *args):
    ngr = len(groups)
    if npack:
        w_refs = args[:ngr]
        w2_refs = None
        rest = args[ngr:]
    else:
        w_refs = args[:ngr]
        w2_refs = args[ngr:2 * ngr]
        rest = args[2 * ngr:]
    out_ref = rest[0]
    m0 = wp + 16
    exn = ch + 2 * wp + 24
    q = ch // wp

    def chunk(c, carry):
        base = pl.multiple_of(c * ch, 8)
        if strip_grid:
            ext = x_ref[0, 0, pl.ds(base, exn), :]
        else:
            ext = x_ref[0, pl.ds(base, exn), :]
        exh, exl = _split(ext)
        acc = None
        for gi, (off0, arels, span) in enumerate(groups):
            eh = exh[off0:off0 + span * wp + ch]
            el = exl[off0:off0 + span * wp + ch]
            lh = jnp.concatenate([eh[a * wp:a * wp + ch] for a in arels], axis=1)
            ll = jnp.concatenate([el[a * wp:a * wp + ch] for a in arels], axis=1)
            lhs = jnp.concatenate([lh, ll], axis=1)
            if npack:
                d = jnp.dot(lhs, w_refs[gi][0], preferred_element_type=f32)
            else:
                d = jnp.dot(lh, w_refs[gi][0], preferred_element_type=f32)
                d = d + jnp.dot(lhs, w2_refs[gi][0], preferred_element_type=f32)
            acc = d if acc is None else acc + d
        if npack:
            acc = acc[:, :nout] + acc[:, nout:]
        acc = acc + b_ref[...]
        if act == 'relu':
            acc = jnp.maximum(acc, 0.0)
        elif act == 'tanh':
            acc = jnp.tanh(acc)
        if mask is not None:
            h_img, w_img = mask
            a3 = acc.reshape(q, wp, nout)
            ti = jax.lax.broadcasted_iota(jnp.int32, (q, wp, 1), 0) + c * q
            ji = jax.lax.broadcasted_iota(jnp.int32, (q, wp, 1), 1)
            if strip_h is not None:
                ti = ti + pl.program_id(1) * strip_h - 1
            ok = ((ti >= 1) & (ti <= h_img) & (ji >= 1) & (ji <= w_img))
            a3 = jnp.where(ok, a3, 0.0)
            acc = a3.reshape(ch, nout)
        if strip_grid:
            out_ref[0, 0, pl.ds(m0 + base, ch), :] = acc
        else:
            out_ref[0, pl.ds(m0 + base, ch), :] = acc
        return carry

    jax.lax.fori_loop(0, nchunks, chunk, 0)


def _stage2(x, wgs, b, wp, mout, ch, act='none', npack=True, mask=None,
            strip_h=None):
    strips = x.ndim == 4
    groups = tuple((off0, tuple(arels), max(arels)) for off0, arels, _ in wgs)
    if npack:
        warrs = [w for _, _, w in wgs]
        w2arrs = []
    else:
        warrs = [w[0] for _, _, w in wgs]
        w2arrs = [w[1] for _, _, w in wgs]
    nout = b.shape[-1]
    body = functools.partial(_stage_body, groups, ch, mout // ch, wp, nout,
                             npack, act, mask, strip_h, strips)
    m0 = wp + 16
    min2 = mout + 2 * m0
    if strips:
        bsz, ns, minr, k = x.shape
        grid = (bsz, ns)
        xspec = pl.BlockSpec((1, 1, minr, k), lambda i, j: (i, j, 0, 0))
        ospec = pl.BlockSpec((1, 1, min2, nout), lambda i, j: (i, j, 0, 0))
        oshape = jax.ShapeDtypeStruct((bsz, ns, min2, nout), f32)
        wspec = lambda t, kk, n: pl.BlockSpec((t, kk, n), lambda i, j: (0, 0, 0))
        bspec = pl.BlockSpec((1, nout), lambda i, j: (0, 0))
        sem = ("parallel", "parallel")
    else:
        g, minr, k = x.shape
        grid = (g,)
        xspec = pl.BlockSpec((1, minr, k), lambda i: (i, 0, 0))
        ospec = pl.BlockSpec((1, min2, nout), lambda i: (i, 0, 0))
        oshape = jax.ShapeDtypeStruct((g, min2, nout), f32)
        wspec = lambda t, kk, n: pl.BlockSpec((t, kk, n), lambda i: (0, 0, 0))
        bspec = pl.BlockSpec((1, nout), lambda i: (0, 0))
        sem = ("parallel",)
    warr_specs = [wspec(1, w.shape[0], w.shape[1]) for w in warrs]
    w2_specs = [wspec(1, w.shape[0], w.shape[1]) for w in w2arrs]
    out = pl.pallas_call(
        body,
        grid=grid,
        in_specs=[xspec, bspec] + warr_specs + w2_specs,
        out_specs=ospec,
        out_shape=oshape,
        compiler_params=pltpu.CompilerParams(dimension_semantics=sem),
    )(x, b, *[w[None] for w in warrs], *[w[None] for w in w2arrs])
    return out


def _round8(n):
    return (n + 7) // 8 * 8


def _mk_conv_wgs(w9, wp, npack):
    ci, co = w9.shape[1], w9.shape[0]
    wtap = w9.transpose(2, 3, 1, 0)
    m0 = wp + 16
    wgs = []
    for dj in range(3):
        whs = jnp.concatenate([wtap[di, dj] for di in range(3)], axis=0)
        wh, wl = _split(whs.astype(f32))
        off0 = m0 - wp + dj - 1
        if npack:
            top = jnp.concatenate([wh, wl], axis=1)
            bot = jnp.concatenate([jnp.zeros_like(wh), wh], axis=1)
            wgs.append((off0, [0, 1, 2], jnp.concatenate([top, bot], axis=0)))
        else:
            w2 = jnp.concatenate([wl, wh], axis=0)
            wgs.append((off0, [0, 1, 2], (wh, w2)))
    return wgs


def _mk_plane_wgs(wtap_list, wp, npack):
    m0 = wp + 16
    wgs = []
    for b in (0, 1):
        taps = [(a, w) for a, bb, w in wtap_list if bb == b]
        if not taps:
            continue
        arels = [a for a, _ in taps]
        whs = jnp.concatenate([w for _, w in taps], axis=0)
        wh, wl = _split(whs)
        off0 = m0 + b
        if npack:
            top = jnp.concatenate([wh, wl], axis=1)
            bot = jnp.concatenate([jnp.zeros_like(wh), wh], axis=1)
            wgs.append((off0, arels, jnp.concatenate([top, bot], axis=0)))
        else:
            w2 = jnp.concatenate([wl, wh], axis=0)
            wgs.append((off0, arels, (wh, w2)))
    return wgs


def _deconv_wtaps(w):
    wt = jnp.flip(w, (2, 3)).transpose(1, 0, 2, 3)
    return wt.transpose(2, 3, 1, 0)


def _plane_tap_sets(wtap):
    return [
        [(0, 0, wtap[1, 1])],
        [(0, 0, wtap[1, 0]), (0, 1, wtap[1, 2])],
        [(0, 0, wtap[0, 1]), (1, 0, wtap[2, 1])],
        [(0, 0, wtap[0, 0]), (0, 1, wtap[0, 2]), (1, 0, wtap[2, 0]),
         (1, 1, wtap[2, 2])],
    ]


def _flat_format(xint, h, w):
    bsz, _, _, c = xint.shape
    wp = _round8(w + 2)
    m0 = wp + 16
    ap = jnp.pad(xint, ((0, 0), (1, 1), (1, wp - w - 1), (0, 0)))
    flat = ap.reshape(bsz, (h + 2) * wp, c)
    return jnp.pad(flat, ((0, 0), (m0, m0), (0, 0))), wp


def _merge_planes(planes, h, w, wp):
    m0 = wp + 16
    outs = []
    for p in planes:
        bsz, _, c = p.shape
        v = p[:, m0:m0 + (h + 2) * wp, :].reshape(bsz, h + 2, wp, c)
        outs.append(v[:, 1:h + 1, 1:w + 1, :])
    p00, p01, p10, p11 = outs
    bsz, _, _, c = p00.shape
    q0 = jnp.stack([p00, p01], 3).reshape(bsz, h, 2 * w, c)
    q1 = jnp.stack([p10, p11], 3).reshape(bsz, h, 2 * w, c)
    return jnp.stack([q0, q1], 2).reshape(bsz, 2 * h, 2 * w, c)


def _deconv_level(x_fmt, w, b, h, w_sp, wp, ch, npack):
    wtap = _deconv_wtaps(w)
    mout = (h + 2) * wp
    planes = []
    for tset in _plane_tap_sets(wtap):
        wgs = _mk_plane_wgs(tset, wp, npack)
        o = _stage2(x_fmt, wgs, b[None, :], wp, mout, ch, act='relu',
                    npack=npack)
        planes.append(o)
    return _merge_planes(planes, h, w_sp, wp)


def kernel(x, We0, be0, We1, be1, Wi, bi, Wd1, bd1, Wc1, bc1, Wd2, bd2,
           Wc2, bc2, Wd3, bd3, Wc3, bc3, Wo, bo):
    bsz = x.shape[0]
    lat_fine = _conv_xla(x, We0, be0, stride=8)
    lat_coarse = _conv_xla(x, We1, be1, stride=16)
    gray = 0.299 * x[:, 0] + 0.587 * x[:, 1] + 0.114 * x[:, 2]
    ent = _entropy_map(gray)
    thr = jnp.quantile(ent.reshape(-1), 0.5)
    grain = (ent > thr).astype(x.dtype)
    coarse_up = jnp.repeat(jnp.repeat(lat_coarse, 2, axis=2), 2, axis=3)
    g = jnp.repeat(jnp.repeat(grain, 2, axis=1), 2, axis=2)[:, None]
    routed = g * lat_fine + (1.0 - g) * coarse_up

    r = routed.transpose(0, 2, 3, 1)
    r = jnp.pad(r, ((0, 0), (0, 0), (0, 0), (0, 4)))
    r_fmt, wp1 = _flat_format(r, 32, 32)
    wi8 = jnp.pad(Wi, ((0, 0), (0, 4), (0, 0), (0, 0)))
    h_fmt = _stage2(r_fmt, _mk_conv_wgs(wi8, wp1, False), bi[None, :], wp1,
                    34 * 40, ch=680, act='none', npack=False, mask=(32, 32))
    d1 = _deconv_level(h_fmt, Wd1, bd1, 32, 32, wp1, ch=680, npack=False)
    d1_fmt, wp2 = _flat_format(d1, 64, 64)
    h1_fmt = _stage2(d1_fmt, _mk_conv_wgs(Wc1, wp2, False), bc1[None, :], wp2,
                     66 * 72, ch=792, act='relu', npack=False, mask=(64, 64))
    d2 = _deconv_level(h1_fmt, Wd2, bd2, 64, 64, wp2, ch=792, npack=True)
    d2_fmt, wp3 = _flat_format(d2, 128, 128)
    h2_fmt = _stage2(d2_fmt, _mk_conv_wgs(Wc2, wp3, True), bc2[None, :], wp3,
                     130 * 136, ch=1768, act='relu', npack=True,
                     mask=(128, 128))
    d3 = _deconv_level(h2_fmt, Wd3, bd3, 128, 128, wp3, ch=1768, npack=True)
    wp4 = 264
    d3p = jnp.pad(d3, ((0, 0), (2, 2), (1, wp4 - 256 - 1), (0, 0)))
    st = jnp.stack([d3p[:, 64 * s:64 * s + 68] for s in range(4)], 1)
    st = st.reshape(bsz, 4, 68 * wp4, 128)
    m04 = wp4 + 16
    st = jnp.pad(st, ((0, 0), (0, 0), (m04, m04), (0, 0)))
    h3_fmt = _stage2(st, _mk_conv_wgs(Wc3, wp4, True), bc3[None, :], wp4,
                     68 * wp4, ch=1056, act='relu', npack=True,
                     mask=(256, 256), strip_h=64)
    wo8 = jnp.pad(Wo, ((0, 5), (0, 0), (0, 0), (0, 0)))
    bo8 = jnp.pad(bo, ((0, 5),))
    rec_fmt = _stage2(h3_fmt, _mk_conv_wgs(wo8, wp4, True), bo8[None, :], wp4,
                      68 * wp4, ch=1056, act='tanh', npack=True)
    rec_v = rec_fmt[:, :, m04:m04 + 68 * wp4, :].reshape(bsz, 4, 68, wp4, 8)
    rec = rec_v[:, :, 2:66, 1:257, :3].reshape(bsz, 256, 256, 3)
    rec = rec.transpose(0, 3, 1, 2)
    return rec, routed, grain, ent


# speedup vs baseline: 2.6714x; 1.4345x over previous
import functools

import jax
import jax.numpy as jnp
from jax.experimental import pallas as pl
from jax.experimental.pallas import tpu as pltpu

f32 = jnp.float32
bf16 = jnp.bfloat16

_DN = ('NCHW', 'OIHW', 'NCHW')
_NBINS = 256
_NPIX = 256
_NPATCH = 256


def _conv_xla(x, w, b, stride=1, pad=1):
    y = jax.lax.conv_general_dilated(x, w, (stride, stride), [(pad, pad), (pad, pad)],
                                     dimension_numbers=_DN)
    return y + b[None, :, None, None]


def _ent_kernel(v_ref, ent_ref, pdf_scr):
    v = v_ref[0]

    def body(k, carry):
        rows = []
        for d in range(8):
            m = k * 8 + d
            b = m.astype(f32) * (1.0 / 255.0)
            t = (v - b) * 100.0
            w = jnp.exp(-0.5 * (t * t))
            rows.append(jnp.sum(w, axis=0, keepdims=True))
        pdf_scr[pl.ds(k, 1)] = jnp.concatenate(rows, axis=0)[None]
        return carry

    jax.lax.fori_loop(0, _NBINS // 8, body, 0)
    pdf = pdf_scr[...].reshape(_NBINS, _NPATCH)
    s = jnp.sum(pdf, axis=0, keepdims=True)
    pn = jnp.maximum(pdf * (1.0 / s), 1e-10)
    ent_ref[...] = -jnp.sum(pn * jnp.log2(pn), axis=0, keepdims=True)[None]


def _entropy_map(gray):
    bsz = gray.shape[0]
    p = gray.reshape(bsz, 16, 16, 16, 16).transpose(0, 2, 4, 1, 3)
    p = p.reshape(bsz, _NPIX, _NPATCH)
    ent = pl.pallas_call(
        _ent_kernel,
        grid=(bsz,),
        in_specs=[pl.BlockSpec((1, _NPIX, _NPATCH), lambda b: (b, 0, 0))],
        out_specs=pl.BlockSpec((1, 1, _NPATCH), lambda b: (b, 0, 0)),
        out_shape=jax.ShapeDtypeStruct((bsz, 1, _NPATCH), jnp.float32),
        scratch_shapes=[pltpu.VMEM((_NBINS // 8, 8, _NPATCH), jnp.float32)],
        compiler_params=pltpu.CompilerParams(dimension_semantics=("parallel",)),
    )(p)
    return ent.reshape(bsz, 16, 16)


def _split(a):
    hi = a.astype(bf16)
    lo = (a - hi.astype(f32)).astype(bf16)
    return hi, lo


def _stage_body(groups, ch, nchunks, wp, nout, npack, act, mask, strip_h,
                strip_grid, in_bf16, out_bf16, x_ref, b_ref, *args):
    ngr = len(groups)
    if npack:
        w_refs = args[:ngr]
        w2_refs = None
        rest = args[ngr:]
    else:
        w_refs = args[:ngr]
        w2_refs = args[ngr:2 * ngr]
        rest = args[2 * ngr:]
    out_ref = rest[0]
    m0 = wp + 16
    exn = ch + 2 * wp + 24
    q = ch // wp

    def chunk(c, carry):
        base = pl.multiple_of(c * ch, 8)
        if strip_grid:
            ext = x_ref[0, 0, pl.ds(base, exn), :]
        else:
            ext = x_ref[0, pl.ds(base, exn), :]
        if in_bf16:
            exh, exl = ext, None
        else:
            exh, exl = _split(ext)
        acc = None
        for gi, (off0, arels, span) in enumerate(groups):
            eh = exh[off0:off0 + span * wp + ch]
            lh = jnp.concatenate([eh[a * wp:a * wp + ch] for a in arels], axis=1)
            if in_bf16:
                lhs = lh
            else:
                el = exl[off0:off0 + span * wp + ch]
                ll = jnp.concatenate([el[a * wp:a * wp + ch] for a in arels], axis=1)
                lhs = jnp.concatenate([lh, ll], axis=1)
            if npack:
                d = jnp.dot(lhs, w_refs[gi][0], preferred_element_type=f32)
            else:
                d = jnp.dot(lh, w_refs[gi][0], preferred_element_type=f32)
                d = d + jnp.dot(lhs, w2_refs[gi][0], preferred_element_type=f32)
            acc = d if acc is None else acc + d
        if npack:
            acc = acc[:, :nout] + acc[:, nout:]
        acc = acc + b_ref[...]
        if act == 'relu':
            acc = jnp.maximum(acc, 0.0)
        elif act == 'tanh':
            acc = jnp.tanh(acc)
        if mask is not None:
            h_img, w_img = mask
            a3 = acc.reshape(q, wp, nout)
            ti = jax.lax.broadcasted_iota(jnp.int32, (q, wp, 1), 0) + c * q
            ji = jax.lax.broadcasted_iota(jnp.int32, (q, wp, 1), 1)
            if strip_h is not None:
                ti = ti + pl.program_id(1) * strip_h - 1
            ok = ((ti >= 1) & (ti <= h_img) & (ji >= 1) & (ji <= w_img))
            a3 = jnp.where(ok, a3, 0.0)
            acc = a3.reshape(ch, nout)
        if out_bf16:
            acc = acc.astype(bf16)
        if strip_grid:
            out_ref[0, 0, pl.ds(m0 + base, ch), :] = acc
        else:
            out_ref[0, pl.ds(m0 + base, ch), :] = acc
        return carry

    jax.lax.fori_loop(0, nchunks, chunk, 0)


def _stage2(x, wgs, b, wp, mout, ch, act='none', npack=True, mask=None,
            strip_h=None, out_bf16=False):
    strips = x.ndim == 4
    groups = tuple((off0, tuple(arels), max(arels)) for off0, arels, _ in wgs)
    if npack:
        warrs = [w for _, _, w in wgs]
        w2arrs = []
    else:
        warrs = [w[0] for _, _, w in wgs]
        w2arrs = [w[1] for _, _, w in wgs]
    nout = b.shape[-1]
    in_bf16 = x.dtype == bf16
    body = functools.partial(_stage_body, groups, ch, mout // ch, wp, nout,
                             npack, act, mask, strip_h, strips, in_bf16,
                             out_bf16)
    m0 = wp + 16
    min2 = mout + 2 * m0
    if strips:
        bsz, ns, minr, k = x.shape
        grid = (bsz, ns)
        xspec = pl.BlockSpec((1, 1, minr, k), lambda i, j: (i, j, 0, 0))
        ospec = pl.BlockSpec((1, 1, min2, nout), lambda i, j: (i, j, 0, 0))
        oshape = jax.ShapeDtypeStruct((bsz, ns, min2, nout), bf16 if out_bf16 else f32)
        wspec = lambda t, kk, n: pl.BlockSpec((t, kk, n), lambda i, j: (0, 0, 0))
        bspec = pl.BlockSpec((1, nout), lambda i, j: (0, 0))
        sem = ("parallel", "parallel")
    else:
        g, minr, k = x.shape
        grid = (g,)
        xspec = pl.BlockSpec((1, minr, k), lambda i: (i, 0, 0))
        ospec = pl.BlockSpec((1, min2, nout), lambda i: (i, 0, 0))
        oshape = jax.ShapeDtypeStruct((g, min2, nout), bf16 if out_bf16 else f32)
        wspec = lambda t, kk, n: pl.BlockSpec((t, kk, n), lambda i: (0, 0, 0))
        bspec = pl.BlockSpec((1, nout), lambda i: (0, 0))
        sem = ("parallel",)
    warr_specs = [wspec(1, w.shape[0], w.shape[1]) for w in warrs]
    w2_specs = [wspec(1, w.shape[0], w.shape[1]) for w in w2arrs]
    out = pl.pallas_call(
        body,
        grid=grid,
        in_specs=[xspec, bspec] + warr_specs + w2_specs,
        out_specs=ospec,
        out_shape=oshape,
        compiler_params=pltpu.CompilerParams(dimension_semantics=sem),
    )(x, b, *[w[None] for w in warrs], *[w[None] for w in w2arrs])
    return out


def _round8(n):
    return (n + 7) // 8 * 8


def _pack_w(wh, wl, mode):
    if mode == 'x2':
        return jnp.concatenate([wh, wl], axis=1)
    if mode == 'x3npack':
        top = jnp.concatenate([wh, wl], axis=1)
        bot = jnp.concatenate([jnp.zeros_like(wh), wh], axis=1)
        return jnp.concatenate([top, bot], axis=0)
    return (wh, jnp.concatenate([wl, wh], axis=0))


def _mk_conv_wgs(w9, wp, mode):
    wtap = w9.transpose(2, 3, 1, 0)
    m0 = wp + 16
    wgs = []
    for dj in range(3):
        whs = jnp.concatenate([wtap[di, dj] for di in range(3)], axis=0)
        wh, wl = _split(whs.astype(f32))
        off0 = m0 - wp + dj - 1
        wgs.append((off0, [0, 1, 2], _pack_w(wh, wl, mode)))
    return wgs


def _mk_plane_wgs(wtap_list, wp, mode):
    m0 = wp + 16
    wgs = []
    for b in (0, 1):
        taps = [(a, w) for a, bb, w in wtap_list if bb == b]
        if not taps:
            continue
        arels = [a for a, _ in taps]
        whs = jnp.concatenate([w for _, w in taps], axis=0)
        wh, wl = _split(whs)
        off0 = m0 + b
        wgs.append((off0, arels, _pack_w(wh, wl, mode)))
    return wgs


def _deconv_wtaps(w):
    wt = jnp.flip(w, (2, 3)).transpose(1, 0, 2, 3)
    return wt.transpose(2, 3, 1, 0)


def _plane_tap_sets(wtap):
    return [
        [(0, 0, wtap[1, 1])],
        [(0, 0, wtap[1, 0]), (0, 1, wtap[1, 2])],
        [(0, 0, wtap[0, 1]), (1, 0, wtap[2, 1])],
        [(0, 0, wtap[0, 0]), (0, 1, wtap[0, 2]), (1, 0, wtap[2, 0]),
         (1, 1, wtap[2, 2])],
    ]


def _flat_format(xint, h, w):
    bsz, _, _, c = xint.shape
    wp = _round8(w + 2)
    m0 = wp + 16
    ap = jnp.pad(xint, ((0, 0), (1, 1), (1, wp - w - 1), (0, 0)))
    flat = ap.reshape(bsz, (h + 2) * wp, c)
    return jnp.pad(flat, ((0, 0), (m0, m0), (0, 0))), wp


def _merge_planes(planes, h, w, wp):
    m0 = wp + 16
    outs = []
    for p in planes:
        bsz, _, c = p.shape
        v = p[:, m0:m0 + (h + 2) * wp, :].reshape(bsz, h + 2, wp, c)
        outs.append(v[:, 1:h + 1, 1:w + 1, :])
    p00, p01, p10, p11 = outs
    bsz, _, _, c = p00.shape
    q0 = jnp.stack([p00, p01], 3).reshape(bsz, h, 2 * w, c)
    q1 = jnp.stack([p10, p11], 3).reshape(bsz, h, 2 * w, c)
    return jnp.stack([q0, q1], 2).reshape(bsz, 2 * h, 2 * w, c)


def _deconv_level(x_fmt, w, b, h, w_sp, wp, ch, mode, out_bf16=False):
    wtap = _deconv_wtaps(w)
    mout = (h + 2) * wp
    planes = []
    for tset in _plane_tap_sets(wtap):
        wgs = _mk_plane_wgs(tset, wp, mode)
        o = _stage2(x_fmt, wgs, b[None, :], wp, mout, ch, act='relu',
                    npack=(mode != 'x3two'), out_bf16=out_bf16)
        planes.append(o)
    return _merge_planes(planes, h, w_sp, wp)


def kernel(x, We0, be0, We1, be1, Wi, bi, Wd1, bd1, Wc1, bc1, Wd2, bd2,
           Wc2, bc2, Wd3, bd3, Wc3, bc3, Wo, bo):
    bsz = x.shape[0]
    lat_fine = _conv_xla(x, We0, be0, stride=8)
    lat_coarse = _conv_xla(x, We1, be1, stride=16)
    gray = 0.299 * x[:, 0] + 0.587 * x[:, 1] + 0.114 * x[:, 2]
    ent = _entropy_map(gray)
    thr = jnp.quantile(ent.reshape(-1), 0.5)
    grain = (ent > thr).astype(x.dtype)
    coarse_up = jnp.repeat(jnp.repeat(lat_coarse, 2, axis=2), 2, axis=3)
    g = jnp.repeat(jnp.repeat(grain, 2, axis=1), 2, axis=2)[:, None]
    routed = g * lat_fine + (1.0 - g) * coarse_up

    r = routed.transpose(0, 2, 3, 1)
    r = jnp.pad(r, ((0, 0), (0, 0), (0, 0), (0, 4)))
    r_fmt, wp1 = _flat_format(r, 32, 32)
    wi8 = jnp.pad(Wi, ((0, 0), (0, 4), (0, 0), (0, 0)))
    h_fmt = _stage2(r_fmt, _mk_conv_wgs(wi8, wp1, 'x3two'), bi[None, :], wp1,
                    34 * 40, ch=680, act='none', npack=False, mask=(32, 32))
    d1 = _deconv_level(h_fmt, Wd1, bd1, 32, 32, wp1, ch=680, mode='x3two')
    d1_fmt, wp2 = _flat_format(d1, 64, 64)
    h1_fmt = _stage2(d1_fmt, _mk_conv_wgs(Wc1, wp2, 'x3two'), bc1[None, :], wp2,
                     66 * 72, ch=792, act='relu', npack=False, mask=(64, 64))
    d2 = _deconv_level(h1_fmt, Wd2, bd2, 64, 64, wp2, ch=792, mode='x3npack',
                       out_bf16=True)
    d2_fmt, wp3 = _flat_format(d2, 128, 128)
    h2_fmt = _stage2(d2_fmt, _mk_conv_wgs(Wc2, wp3, 'x2'), bc2[None, :], wp3,
                     130 * 136, ch=1768, act='relu', npack=True,
                     mask=(128, 128), out_bf16=True)
    d3 = _deconv_level(h2_fmt, Wd3, bd3, 128, 128, wp3, ch=1768, mode='x2',
                       out_bf16=True)
    wp4 = 264
    d3p = jnp.pad(d3, ((0, 0), (2, 2), (1, wp4 - 256 - 1), (0, 0)))
    st = jnp.stack([d3p[:, 64 * s:64 * s + 68] for s in range(4)], 1)
    st = st.reshape(bsz, 4, 68 * wp4, 128)
    m04 = wp4 + 16
    st = jnp.pad(st, ((0, 0), (0, 0), (m04, m04), (0, 0)))
    h3_fmt = _stage2(st, _mk_conv_wgs(Wc3, wp4, 'x2'), bc3[None, :], wp4,
                     68 * wp4, ch=1056, act='relu', npack=True,
                     mask=(256, 256), strip_h=64, out_bf16=True)
    wo8 = jnp.pad(Wo, ((0, 5), (0, 0), (0, 0), (0, 0)))
    bo8 = jnp.pad(bo, ((0, 5),))
    rec_fmt = _stage2(h3_fmt, _mk_conv_wgs(wo8, wp4, 'x2'), bo8[None, :], wp4,
                      68 * wp4, ch=1056, act='tanh', npack=True)
    rec_v = rec_fmt[:, :, m04:m04 + 68 * wp4, :].reshape(bsz, 4, 68, wp4, 8)
    rec = rec_v[:, :, 2:66, 1:257, :3].reshape(bsz, 256, 256, 3)
    rec = rec.transpose(0, 3, 1, 2)
    return rec, routed, grain, ent


# speedup vs baseline: 2.7257x; 1.0203x over previous
import functools

import jax
import jax.numpy as jnp
from jax.experimental import pallas as pl
from jax.experimental.pallas import tpu as pltpu

f32 = jnp.float32
bf16 = jnp.bfloat16

_DN = ('NCHW', 'OIHW', 'NCHW')
_NBINS = 256
_NPIX = 256
_NPATCH = 256


def _conv_xla(x, w, b, stride=1, pad=1):
    y = jax.lax.conv_general_dilated(x, w, (stride, stride), [(pad, pad), (pad, pad)],
                                     dimension_numbers=_DN)
    return y + b[None, :, None, None]


def _ent_kernel(v_ref, ent_ref, pdf_scr):
    v = v_ref[0]

    def body(k, carry):
        rows = []
        for d in range(8):
            m = k * 8 + d
            b = m.astype(f32) * (1.0 / 255.0)
            t = (v - b) * 100.0
            w = jnp.exp(-0.5 * (t * t))
            rows.append(jnp.sum(w, axis=0, keepdims=True))
        pdf_scr[pl.ds(k, 1)] = jnp.concatenate(rows, axis=0)[None]
        return carry

    jax.lax.fori_loop(0, _NBINS // 8, body, 0)
    pdf = pdf_scr[...].reshape(_NBINS, _NPATCH)
    s = jnp.sum(pdf, axis=0, keepdims=True)
    pn = jnp.maximum(pdf * (1.0 / s), 1e-10)
    ent_ref[...] = -jnp.sum(pn * jnp.log2(pn), axis=0, keepdims=True)[None]


def _entropy_map(gray):
    bsz = gray.shape[0]
    p = gray.reshape(bsz, 16, 16, 16, 16).transpose(0, 2, 4, 1, 3)
    p = p.reshape(bsz, _NPIX, _NPATCH)
    ent = pl.pallas_call(
        _ent_kernel,
        grid=(bsz,),
        in_specs=[pl.BlockSpec((1, _NPIX, _NPATCH), lambda b: (b, 0, 0))],
        out_specs=pl.BlockSpec((1, 1, _NPATCH), lambda b: (b, 0, 0)),
        out_shape=jax.ShapeDtypeStruct((bsz, 1, _NPATCH), jnp.float32),
        scratch_shapes=[pltpu.VMEM((_NBINS // 8, 8, _NPATCH), jnp.float32)],
        compiler_params=pltpu.CompilerParams(dimension_semantics=("parallel",)),
    )(p)
    return ent.reshape(bsz, 16, 16)


def _split(a):
    hi = a.astype(bf16)
    lo = (a - hi.astype(f32)).astype(bf16)
    return hi, lo


def _stage_body(groups, ch, nchunks, wp, nout, npack, act, mask, strip_h,
                strip_grid, in_bf16, out_bf16, x_ref, b_ref, *args):
    ngr = len(groups)
    if npack:
        w_refs = args[:ngr]
        w2_refs = None
        rest = args[ngr:]
    else:
        w_refs = args[:ngr]
        w2_refs = args[ngr:2 * ngr]
        rest = args[2 * ngr:]
    out_ref = rest[0]
    m0 = wp + 16
    exn = ch + 2 * wp + 24
    q = ch // wp

    def chunk(c, carry):
        base = pl.multiple_of(c * ch, 8)
        if strip_grid:
            ext = x_ref[0, 0, pl.ds(base, exn), :]
        else:
            ext = x_ref[0, pl.ds(base, exn), :]
        if in_bf16:
            exh, exl = ext, None
        else:
            exh, exl = _split(ext)
        acc = None
        for gi, (off0, arels, span) in enumerate(groups):
            eh = exh[off0:off0 + span * wp + ch]
            lh = jnp.concatenate([eh[a * wp:a * wp + ch] for a in arels], axis=1)
            if in_bf16:
                lhs = lh
            else:
                el = exl[off0:off0 + span * wp + ch]
                ll = jnp.concatenate([el[a * wp:a * wp + ch] for a in arels], axis=1)
                lhs = jnp.concatenate([lh, ll], axis=1)
            if npack:
                d = jnp.dot(lhs, w_refs[gi][0], preferred_element_type=f32)
            else:
                d = jnp.dot(lh, w_refs[gi][0], preferred_element_type=f32)
                d = d + jnp.dot(lhs, w2_refs[gi][0], preferred_element_type=f32)
            acc = d if acc is None else acc + d
        if npack:
            acc = acc[:, :nout] + acc[:, nout:]
        acc = acc + b_ref[...]
        if act == 'relu':
            acc = jnp.maximum(acc, 0.0)
        elif act == 'tanh':
            acc = jnp.tanh(acc)
        if mask is not None:
            h_img, w_img = mask
            a3 = acc.reshape(q, wp, nout)
            ti = jax.lax.broadcasted_iota(jnp.int32, (q, wp, 1), 0) + c * q
            ji = jax.lax.broadcasted_iota(jnp.int32, (q, wp, 1), 1)
            if strip_h is not None:
                ti = ti + pl.program_id(1) * strip_h - 1
            ok = ((ti >= 1) & (ti <= h_img) & (ji >= 1) & (ji <= w_img))
            a3 = jnp.where(ok, a3, 0.0)
            acc = a3.reshape(ch, nout)
        if out_bf16:
            acc = acc.astype(bf16)
        if strip_grid:
            out_ref[0, 0, pl.ds(m0 + base, ch), :] = acc
        else:
            out_ref[0, pl.ds(m0 + base, ch), :] = acc
        return carry

    jax.lax.fori_loop(0, nchunks, chunk, 0)


def _stage2(x, wgs, b, wp, mout, ch, act='none', npack=True, mask=None,
            strip_h=None, out_bf16=False):
    strips = x.ndim == 4
    groups = tuple((off0, tuple(arels), max(arels)) for off0, arels, _ in wgs)
    if npack:
        warrs = [w for _, _, w in wgs]
        w2arrs = []
    else:
        warrs = [w[0] for _, _, w in wgs]
        w2arrs = [w[1] for _, _, w in wgs]
    nout = b.shape[-1]
    in_bf16 = x.dtype == bf16
    body = functools.partial(_stage_body, groups, ch, mout // ch, wp, nout,
                             npack, act, mask, strip_h, strips, in_bf16,
                             out_bf16)
    m0 = wp + 16
    min2 = mout + 2 * m0
    if strips:
        bsz, ns, minr, k = x.shape
        grid = (bsz, ns)
        xspec = pl.BlockSpec((1, 1, minr, k), lambda i, j: (i, j, 0, 0))
        ospec = pl.BlockSpec((1, 1, min2, nout), lambda i, j: (i, j, 0, 0))
        oshape = jax.ShapeDtypeStruct((bsz, ns, min2, nout), bf16 if out_bf16 else f32)
        wspec = lambda t, kk, n: pl.BlockSpec((t, kk, n), lambda i, j: (0, 0, 0))
        bspec = pl.BlockSpec((1, nout), lambda i, j: (0, 0))
        sem = ("parallel", "parallel")
    else:
        g, minr, k = x.shape
        grid = (g,)
        xspec = pl.BlockSpec((1, minr, k), lambda i: (i, 0, 0))
        ospec = pl.BlockSpec((1, min2, nout), lambda i: (i, 0, 0))
        oshape = jax.ShapeDtypeStruct((g, min2, nout), bf16 if out_bf16 else f32)
        wspec = lambda t, kk, n: pl.BlockSpec((t, kk, n), lambda i: (0, 0, 0))
        bspec = pl.BlockSpec((1, nout), lambda i: (0, 0))
        sem = ("parallel",)
    warr_specs = [wspec(1, w.shape[0], w.shape[1]) for w in warrs]
    w2_specs = [wspec(1, w.shape[0], w.shape[1]) for w in w2arrs]
    out = pl.pallas_call(
        body,
        grid=grid,
        in_specs=[xspec, bspec] + warr_specs + w2_specs,
        out_specs=ospec,
        out_shape=oshape,
        compiler_params=pltpu.CompilerParams(dimension_semantics=sem),
    )(x, b, *[w[None] for w in warrs], *[w[None] for w in w2arrs])
    return out


def _round8(n):
    return (n + 7) // 8 * 8


def _pack_w(wh, wl, mode):
    if mode == 'x2':
        return jnp.concatenate([wh, wl], axis=1)
    if mode == 'x3npack':
        top = jnp.concatenate([wh, wl], axis=1)
        bot = jnp.concatenate([jnp.zeros_like(wh), wh], axis=1)
        return jnp.concatenate([top, bot], axis=0)
    return (wh, jnp.concatenate([wl, wh], axis=0))


def _mk_conv_wgs(w9, wp, mode):
    wtap = w9.transpose(2, 3, 1, 0)
    m0 = wp + 16
    wgs = []
    for dj in range(3):
        whs = jnp.concatenate([wtap[di, dj] for di in range(3)], axis=0)
        wh, wl = _split(whs.astype(f32))
        off0 = m0 - wp + dj - 1
        wgs.append((off0, [0, 1, 2], _pack_w(wh, wl, mode)))
    return wgs


def _mk_plane_wgs(wtap_list, wp, mode):
    m0 = wp + 16
    wgs = []
    for b in (0, 1):
        taps = [(a, w) for a, bb, w in wtap_list if bb == b]
        if not taps:
            continue
        arels = [a for a, _ in taps]
        whs = jnp.concatenate([w for _, w in taps], axis=0)
        wh, wl = _split(whs)
        off0 = m0 + b
        wgs.append((off0, arels, _pack_w(wh, wl, mode)))
    return wgs


def _deconv_wtaps(w):
    wt = jnp.flip(w, (2, 3)).transpose(1, 0, 2, 3)
    return wt.transpose(2, 3, 1, 0)


def _plane_tap_sets(wtap):
    return [
        [(0, 0, wtap[1, 1])],
        [(0, 0, wtap[1, 0]), (0, 1, wtap[1, 2])],
        [(0, 0, wtap[0, 1]), (1, 0, wtap[2, 1])],
        [(0, 0, wtap[0, 0]), (0, 1, wtap[0, 2]), (1, 0, wtap[2, 0]),
         (1, 1, wtap[2, 2])],
    ]


def _flat_format(xint, h, w):
    bsz, _, _, c = xint.shape
    wp = _round8(w + 2)
    m0 = wp + 16
    ap = jnp.pad(xint, ((0, 0), (1, 1), (1, wp - w - 1), (0, 0)))
    flat = ap.reshape(bsz, (h + 2) * wp, c)
    return jnp.pad(flat, ((0, 0), (m0, m0), (0, 0))), wp


def _merge_planes(planes, h, w, wp):
    m0 = wp + 16
    outs = []
    for p in planes:
        bsz, _, c = p.shape
        v = p[:, m0:m0 + (h + 2) * wp, :].reshape(bsz, h + 2, wp, c)
        outs.append(v[:, 1:h + 1, 1:w + 1, :])
    p00, p01, p10, p11 = outs
    bsz, _, _, c = p00.shape
    q0 = jnp.stack([p00, p01], 3).reshape(bsz, h, 2 * w, c)
    q1 = jnp.stack([p10, p11], 3).reshape(bsz, h, 2 * w, c)
    return jnp.stack([q0, q1], 2).reshape(bsz, 2 * h, 2 * w, c)


def _planes_body(plane_groups, ch, nchunks, wp, nout, npack, in_bf16,
                 out_bf16, x_ref, b_ref, *args):
    nw = sum(len(g) for g in plane_groups)
    w_refs = args[:nw]
    if npack:
        w2_refs = None
        out_refs = args[nw:nw + 4]
    else:
        w2_refs = args[nw:2 * nw]
        out_refs = args[2 * nw:2 * nw + 4]
    m0 = wp + 16
    exn = ch + 2 * wp + 24

    def chunk(c, carry):
        base = pl.multiple_of(c * ch, 8)
        ext = x_ref[0, pl.ds(base, exn), :]
        if in_bf16:
            exh, exl = ext, None
        else:
            exh, exl = _split(ext)
        wi = 0
        for pi, groups in enumerate(plane_groups):
            acc = None
            for (off0, arels, span) in groups:
                eh = exh[off0:off0 + span * wp + ch]
                lh = jnp.concatenate([eh[a * wp:a * wp + ch] for a in arels],
                                     axis=1)
                if in_bf16:
                    lhs = lh
                else:
                    el = exl[off0:off0 + span * wp + ch]
                    ll = jnp.concatenate([el[a * wp:a * wp + ch] for a in arels],
                                         axis=1)
                    lhs = jnp.concatenate([lh, ll], axis=1)
                if npack:
                    d = jnp.dot(lhs, w_refs[wi][0], preferred_element_type=f32)
                else:
                    d = jnp.dot(lh, w_refs[wi][0], preferred_element_type=f32)
                    d = d + jnp.dot(lhs, w2_refs[wi][0],
                                    preferred_element_type=f32)
                wi += 1
                acc = d if acc is None else acc + d
            if npack:
                acc = acc[:, :nout] + acc[:, nout:]
            acc = jnp.maximum(acc + b_ref[...], 0.0)
            if out_bf16:
                acc = acc.astype(bf16)
            out_refs[pi][0, pl.ds(m0 + base, ch), :] = acc
        return carry

    jax.lax.fori_loop(0, nchunks, chunk, 0)


def _deconv_level(x_fmt, w, b, h, w_sp, wp, ch, mode, out_bf16=False):
    wtap = _deconv_wtaps(w)
    mout = (h + 2) * wp
    npack = mode != 'x3two'
    plane_wgs = [_mk_plane_wgs(tset, wp, mode)
                 for tset in _plane_tap_sets(wtap)]
    plane_groups = tuple(tuple((off0, tuple(arels), max(arels))
                               for off0, arels, _ in wgs)
                         for wgs in plane_wgs)
    if npack:
        warrs = [wg[2] for wgs in plane_wgs for wg in wgs]
        w2arrs = []
    else:
        warrs = [wg[2][0] for wgs in plane_wgs for wg in wgs]
        w2arrs = [wg[2][1] for wgs in plane_wgs for wg in wgs]
    nout = b.shape[-1]
    in_bf16 = x_fmt.dtype == bf16
    body = functools.partial(_planes_body, plane_groups, ch, mout // ch, wp,
                             nout, npack, in_bf16, out_bf16)
    m0 = wp + 16
    min2 = mout + 2 * m0
    g, minr, k = x_fmt.shape
    wspec = lambda kk, n: pl.BlockSpec((1, kk, n), lambda i: (0, 0, 0))
    odt = bf16 if out_bf16 else f32
    planes = pl.pallas_call(
        body,
        grid=(g,),
        in_specs=([pl.BlockSpec((1, minr, k), lambda i: (i, 0, 0)),
                   pl.BlockSpec((1, nout), lambda i: (0, 0))]
                  + [wspec(w_.shape[0], w_.shape[1]) for w_ in warrs]
                  + [wspec(w_.shape[0], w_.shape[1]) for w_ in w2arrs]),
        out_specs=[pl.BlockSpec((1, min2, nout), lambda i: (i, 0, 0))] * 4,
        out_shape=[jax.ShapeDtypeStruct((g, min2, nout), odt)] * 4,
        compiler_params=pltpu.CompilerParams(dimension_semantics=("parallel",)),
    )(x_fmt, b[None, :], *[w_[None] for w_ in warrs],
      *[w_[None] for w_ in w2arrs])
    return _merge_planes(planes, h, w_sp, wp)


def kernel(x, We0, be0, We1, be1, Wi, bi, Wd1, bd1, Wc1, bc1, Wd2, bd2,
           Wc2, bc2, Wd3, bd3, Wc3, bc3, Wo, bo):
    bsz = x.shape[0]
    lat_fine = _conv_xla(x, We0, be0, stride=8)
    lat_coarse = _conv_xla(x, We1, be1, stride=16)
    gray = 0.299 * x[:, 0] + 0.587 * x[:, 1] + 0.114 * x[:, 2]
    ent = _entropy_map(gray)
    thr = jnp.quantile(ent.reshape(-1), 0.5)
    grain = (ent > thr).astype(x.dtype)
    coarse_up = jnp.repeat(jnp.repeat(lat_coarse, 2, axis=2), 2, axis=3)
    g = jnp.repeat(jnp.repeat(grain, 2, axis=1), 2, axis=2)[:, None]
    routed = g * lat_fine + (1.0 - g) * coarse_up

    r = routed.transpose(0, 2, 3, 1)
    r = jnp.pad(r, ((0, 0), (0, 0), (0, 0), (0, 4)))
    r_fmt, wp1 = _flat_format(r, 32, 32)
    wi8 = jnp.pad(Wi, ((0, 0), (0, 4), (0, 0), (0, 0)))
    h_fmt = _stage2(r_fmt, _mk_conv_wgs(wi8, wp1, 'x3two'), bi[None, :], wp1,
                    34 * 40, ch=680, act='none', npack=False, mask=(32, 32))
    d1 = _deconv_level(h_fmt, Wd1, bd1, 32, 32, wp1, ch=680, mode='x3two')
    d1_fmt, wp2 = _flat_format(d1, 64, 64)
    h1_fmt = _stage2(d1_fmt, _mk_conv_wgs(Wc1, wp2, 'x3two'), bc1[None, :], wp2,
                     66 * 72, ch=792, act='relu', npack=False, mask=(64, 64))
    d2 = _deconv_level(h1_fmt, Wd2, bd2, 64, 64, wp2, ch=792, mode='x3npack',
                       out_bf16=True)
    d2_fmt, wp3 = _flat_format(d2, 128, 128)
    h2_fmt = _stage2(d2_fmt, _mk_conv_wgs(Wc2, wp3, 'x2'), bc2[None, :], wp3,
                     130 * 136, ch=1768, act='relu', npack=True,
                     mask=(128, 128), out_bf16=True)
    d3 = _deconv_level(h2_fmt, Wd3, bd3, 128, 128, wp3, ch=1768, mode='x2',
                       out_bf16=True)
    wp4 = 264
    d3p = jnp.pad(d3, ((0, 0), (2, 2), (1, wp4 - 256 - 1), (0, 0)))
    st = jnp.stack([d3p[:, 64 * s:64 * s + 68] for s in range(4)], 1)
    st = st.reshape(bsz, 4, 68 * wp4, 128)
    m04 = wp4 + 16
    st = jnp.pad(st, ((0, 0), (0, 0), (m04, m04), (0, 0)))
    h3_fmt = _stage2(st, _mk_conv_wgs(Wc3, wp4, 'x2'), bc3[None, :], wp4,
                     68 * wp4, ch=1056, act='relu', npack=True,
                     mask=(256, 256), strip_h=64, out_bf16=True)
    wo8 = jnp.pad(Wo, ((0, 5), (0, 0), (0, 0), (0, 0)))
    bo8 = jnp.pad(bo, ((0, 5),))
    rec_fmt = _stage2(h3_fmt, _mk_conv_wgs(wo8, wp4, 'x2'), bo8[None, :], wp4,
                      68 * wp4, ch=1056, act='tanh', npack=True)
    rec_v = rec_fmt[:, :, m04:m04 + 68 * wp4, :].reshape(bsz, 4, 68, wp4, 8)
    rec = rec_v[:, :, 2:66, 1:257, :3].reshape(bsz, 256, 256, 3)
    rec = rec.transpose(0, 3, 1, 2)
    return rec, routed, grain, ent


# speedup vs baseline: 2.7379x; 1.0045x over previous
import functools

import jax
import jax.numpy as jnp
from jax.experimental import pallas as pl
from jax.experimental.pallas import tpu as pltpu

f32 = jnp.float32
bf16 = jnp.bfloat16

_DN = ('NCHW', 'OIHW', 'NCHW')
_NBINS = 256
_NPIX = 256
_NPATCH = 256


def _conv_xla(x, w, b, stride=1, pad=1):
    y = jax.lax.conv_general_dilated(x, w, (stride, stride), [(pad, pad), (pad, pad)],
                                     dimension_numbers=_DN)
    return y + b[None, :, None, None]


def _ent_kernel(v_ref, ent_ref, pdf_scr):
    v = v_ref[0]

    def body(k, carry):
        rows = []
        for d in range(8):
            m = k * 8 + d
            b = m.astype(f32) * (1.0 / 255.0)
            t = (v - b) * 100.0
            w = jnp.exp(-0.5 * (t * t))
            rows.append(jnp.sum(w, axis=0, keepdims=True))
        pdf_scr[pl.ds(k, 1)] = jnp.concatenate(rows, axis=0)[None]
        return carry

    jax.lax.fori_loop(0, _NBINS // 8, body, 0)
    pdf = pdf_scr[...].reshape(_NBINS, _NPATCH)
    s = jnp.sum(pdf, axis=0, keepdims=True)
    pn = jnp.maximum(pdf * (1.0 / s), 1e-10)
    ent_ref[...] = -jnp.sum(pn * jnp.log2(pn), axis=0, keepdims=True)[None]


def _entropy_map(gray):
    bsz = gray.shape[0]
    p = gray.reshape(bsz, 16, 16, 16, 16).transpose(0, 2, 4, 1, 3)
    p = p.reshape(bsz, _NPIX, _NPATCH)
    ent = pl.pallas_call(
        _ent_kernel,
        grid=(bsz,),
        in_specs=[pl.BlockSpec((1, _NPIX, _NPATCH), lambda b: (b, 0, 0))],
        out_specs=pl.BlockSpec((1, 1, _NPATCH), lambda b: (b, 0, 0)),
        out_shape=jax.ShapeDtypeStruct((bsz, 1, _NPATCH), jnp.float32),
        scratch_shapes=[pltpu.VMEM((_NBINS // 8, 8, _NPATCH), jnp.float32)],
        compiler_params=pltpu.CompilerParams(dimension_semantics=("parallel",)),
    )(p)
    return ent.reshape(bsz, 16, 16)


def _split(a):
    hi = a.astype(bf16)
    lo = (a - hi.astype(f32)).astype(bf16)
    return hi, lo


def _stage_body(groups, ch, nchunks, wp, nout, npack, act, mask, strip_h,
                strip_grid, in_bf16, out_bf16, x_ref, b_ref, *args):
    ngr = len(groups)
    if npack:
        w_refs = args[:ngr]
        w2_refs = None
        rest = args[ngr:]
    else:
        w_refs = args[:ngr]
        w2_refs = args[ngr:2 * ngr]
        rest = args[2 * ngr:]
    out_ref = rest[0]
    m0 = wp + 16
    exn = ch + 2 * wp + 24
    q = ch // wp

    def chunk(c, carry):
        base = pl.multiple_of(c * ch, 8)
        if strip_grid:
            ext = x_ref[0, 0, pl.ds(base, exn), :]
        else:
            ext = x_ref[0, pl.ds(base, exn), :]
        if in_bf16:
            exh, exl = ext, None
        else:
            exh, exl = _split(ext)
        acc = None
        for gi, (off0, arels, span) in enumerate(groups):
            eh = exh[off0:off0 + span * wp + ch]
            lh = jnp.concatenate([eh[a * wp:a * wp + ch] for a in arels], axis=1)
            if in_bf16:
                lhs = lh
            else:
                el = exl[off0:off0 + span * wp + ch]
                ll = jnp.concatenate([el[a * wp:a * wp + ch] for a in arels], axis=1)
                lhs = jnp.concatenate([lh, ll], axis=1)
            if npack:
                d = jnp.dot(lhs, w_refs[gi][0], preferred_element_type=f32)
            else:
                d = jnp.dot(lh, w_refs[gi][0], preferred_element_type=f32)
                d = d + jnp.dot(lhs, w2_refs[gi][0], preferred_element_type=f32)
            acc = d if acc is None else acc + d
        if npack:
            acc = acc[:, :nout] + acc[:, nout:]
        acc = acc + b_ref[...]
        if act == 'relu':
            acc = jnp.maximum(acc, 0.0)
        elif act == 'tanh':
            acc = jnp.tanh(acc)
        if mask is not None:
            h_img, w_img = mask
            a3 = acc.reshape(q, wp, nout)
            ti = jax.lax.broadcasted_iota(jnp.int32, (q, wp, 1), 0) + c * q
            ji = jax.lax.broadcasted_iota(jnp.int32, (q, wp, 1), 1)
            if strip_h is not None:
                ti = ti + pl.program_id(1) * strip_h - 1
            ok = ((ti >= 1) & (ti <= h_img) & (ji >= 1) & (ji <= w_img))
            a3 = jnp.where(ok, a3, 0.0)
            acc = a3.reshape(ch, nout)
        if out_bf16:
            acc = acc.astype(bf16)
        if strip_grid:
            out_ref[0, 0, pl.ds(m0 + base, ch), :] = acc
        else:
            out_ref[0, pl.ds(m0 + base, ch), :] = acc
        return carry

    jax.lax.fori_loop(0, nchunks, chunk, 0)


def _stage2(x, wgs, b, wp, mout, ch, act='none', npack=True, mask=None,
            strip_h=None, out_bf16=False):
    strips = x.ndim == 4
    groups = tuple((off0, tuple(arels), max(arels)) for off0, arels, _ in wgs)
    if npack:
        warrs = [w for _, _, w in wgs]
        w2arrs = []
    else:
        warrs = [w[0] for _, _, w in wgs]
        w2arrs = [w[1] for _, _, w in wgs]
    nout = b.shape[-1]
    in_bf16 = x.dtype == bf16
    body = functools.partial(_stage_body, groups, ch, mout // ch, wp, nout,
                             npack, act, mask, strip_h, strips, in_bf16,
                             out_bf16)
    m0 = wp + 16
    min2 = mout + 2 * m0
    if strips:
        bsz, ns, minr, k = x.shape
        grid = (bsz, ns)
        xspec = pl.BlockSpec((1, 1, minr, k), lambda i, j: (i, j, 0, 0))
        ospec = pl.BlockSpec((1, 1, min2, nout), lambda i, j: (i, j, 0, 0))
        oshape = jax.ShapeDtypeStruct((bsz, ns, min2, nout), bf16 if out_bf16 else f32)
        wspec = lambda t, kk, n: pl.BlockSpec((t, kk, n), lambda i, j: (0, 0, 0))
        bspec = pl.BlockSpec((1, nout), lambda i, j: (0, 0))
        sem = ("parallel", "parallel")
    else:
        g, minr, k = x.shape
        grid = (g,)
        xspec = pl.BlockSpec((1, minr, k), lambda i: (i, 0, 0))
        ospec = pl.BlockSpec((1, min2, nout), lambda i: (i, 0, 0))
        oshape = jax.ShapeDtypeStruct((g, min2, nout), bf16 if out_bf16 else f32)
        wspec = lambda t, kk, n: pl.BlockSpec((t, kk, n), lambda i: (0, 0, 0))
        bspec = pl.BlockSpec((1, nout), lambda i: (0, 0))
        sem = ("parallel",)
    warr_specs = [wspec(1, w.shape[0], w.shape[1]) for w in warrs]
    w2_specs = [wspec(1, w.shape[0], w.shape[1]) for w in w2arrs]
    out = pl.pallas_call(
        body,
        grid=grid,
        in_specs=[xspec, bspec] + warr_specs + w2_specs,
        out_specs=ospec,
        out_shape=oshape,
        compiler_params=pltpu.CompilerParams(dimension_semantics=sem),
    )(x, b, *[w[None] for w in warrs], *[w[None] for w in w2arrs])
    return out


def _round8(n):
    return (n + 7) // 8 * 8


def _pack_w(wh, wl, mode):
    if mode == 'x2':
        return jnp.concatenate([wh, wl], axis=1)
    if mode == 'x3npack':
        top = jnp.concatenate([wh, wl], axis=1)
        bot = jnp.concatenate([jnp.zeros_like(wh), wh], axis=1)
        return jnp.concatenate([top, bot], axis=0)
    return (wh, jnp.concatenate([wl, wh], axis=0))


def _mk_conv_wgs(w9, wp, mode):
    wtap = w9.transpose(2, 3, 1, 0)
    m0 = wp + 16
    wgs = []
    for dj in range(3):
        whs = jnp.concatenate([wtap[di, dj] for di in range(3)], axis=0)
        wh, wl = _split(whs.astype(f32))
        off0 = m0 - wp + dj - 1
        wgs.append((off0, [0, 1, 2], _pack_w(wh, wl, mode)))
    return wgs


def _mk_plane_wgs(wtap_list, wp, mode):
    m0 = wp + 16
    wgs = []
    for b in (0, 1):
        taps = [(a, w) for a, bb, w in wtap_list if bb == b]
        if not taps:
            continue
        arels = [a for a, _ in taps]
        whs = jnp.concatenate([w for _, w in taps], axis=0)
        wh, wl = _split(whs)
        off0 = m0 + b
        wgs.append((off0, arels, _pack_w(wh, wl, mode)))
    return wgs


def _deconv_wtaps(w):
    wt = jnp.flip(w, (2, 3)).transpose(1, 0, 2, 3)
    return wt.transpose(2, 3, 1, 0)


def _plane_tap_sets(wtap):
    return [
        [(0, 0, wtap[1, 1])],
        [(0, 0, wtap[1, 0]), (0, 1, wtap[1, 2])],
        [(0, 0, wtap[0, 1]), (1, 0, wtap[2, 1])],
        [(0, 0, wtap[0, 0]), (0, 1, wtap[0, 2]), (1, 0, wtap[2, 0]),
         (1, 1, wtap[2, 2])],
    ]


def _flat_format(xint, h, w):
    bsz, _, _, c = xint.shape
    wp = _round8(w + 2)
    m0 = wp + 16
    ap = jnp.pad(xint, ((0, 0), (1, 1), (1, wp - w - 1), (0, 0)))
    flat = ap.reshape(bsz, (h + 2) * wp, c)
    return jnp.pad(flat, ((0, 0), (m0, m0), (0, 0))), wp


def _merge_planes(planes, h, w, wp):
    m0 = wp + 16
    outs = []
    for p in planes:
        bsz, _, c = p.shape
        v = p[:, m0:m0 + (h + 2) * wp, :].reshape(bsz, h + 2, wp, c)
        outs.append(v[:, 1:h + 1, 1:w + 1, :])
    p00, p01, p10, p11 = outs
    bsz, _, _, c = p00.shape
    q0 = jnp.stack([p00, p01], 3).reshape(bsz, h, 2 * w, c)
    q1 = jnp.stack([p10, p11], 3).reshape(bsz, h, 2 * w, c)
    return jnp.stack([q0, q1], 2).reshape(bsz, 2 * h, 2 * w, c)


def _planes_body(plane_groups, ch, nchunks, wp, nout, npack, in_bf16,
                 out_bf16, x_ref, b_ref, *args):
    nw = sum(len(g) for g in plane_groups)
    w_refs = args[:nw]
    if npack:
        w2_refs = None
        out_refs = args[nw:nw + 4]
    else:
        w2_refs = args[nw:2 * nw]
        out_refs = args[2 * nw:2 * nw + 4]
    m0 = wp + 16
    exn = ch + 2 * wp + 24

    def chunk(c, carry):
        base = pl.multiple_of(c * ch, 8)
        ext = x_ref[0, pl.ds(base, exn), :]
        if in_bf16:
            exh, exl = ext, None
        else:
            exh, exl = _split(ext)
        wi = 0
        for pi, groups in enumerate(plane_groups):
            acc = None
            for (off0, arels, span) in groups:
                eh = exh[off0:off0 + span * wp + ch]
                lh = jnp.concatenate([eh[a * wp:a * wp + ch] for a in arels],
                                     axis=1)
                if in_bf16:
                    lhs = lh
                else:
                    el = exl[off0:off0 + span * wp + ch]
                    ll = jnp.concatenate([el[a * wp:a * wp + ch] for a in arels],
                                         axis=1)
                    lhs = jnp.concatenate([lh, ll], axis=1)
                if npack:
                    d = jnp.dot(lhs, w_refs[wi][0], preferred_element_type=f32)
                else:
                    d = jnp.dot(lh, w_refs[wi][0], preferred_element_type=f32)
                    d = d + jnp.dot(lhs, w2_refs[wi][0],
                                    preferred_element_type=f32)
                wi += 1
                acc = d if acc is None else acc + d
            if npack:
                acc = acc[:, :nout] + acc[:, nout:]
            acc = jnp.maximum(acc + b_ref[...], 0.0)
            if out_bf16:
                acc = acc.astype(bf16)
            out_refs[pi][0, pl.ds(m0 + base, ch), :] = acc
        return carry

    jax.lax.fori_loop(0, nchunks, chunk, 0)


def _deconv_level(x_fmt, w, b, h, w_sp, wp, ch, mode, out_bf16=False):
    wtap = _deconv_wtaps(w)
    mout = (h + 2) * wp
    npack = mode != 'x3two'
    plane_wgs = [_mk_plane_wgs(tset, wp, mode)
                 for tset in _plane_tap_sets(wtap)]
    plane_groups = tuple(tuple((off0, tuple(arels), max(arels))
                               for off0, arels, _ in wgs)
                         for wgs in plane_wgs)
    if npack:
        warrs = [wg[2] for wgs in plane_wgs for wg in wgs]
        w2arrs = []
    else:
        warrs = [wg[2][0] for wgs in plane_wgs for wg in wgs]
        w2arrs = [wg[2][1] for wgs in plane_wgs for wg in wgs]
    nout = b.shape[-1]
    in_bf16 = x_fmt.dtype == bf16
    body = functools.partial(_planes_body, plane_groups, ch, mout // ch, wp,
                             nout, npack, in_bf16, out_bf16)
    m0 = wp + 16
    min2 = mout + 2 * m0
    g, minr, k = x_fmt.shape
    wspec = lambda kk, n: pl.BlockSpec((1, kk, n), lambda i: (0, 0, 0))
    odt = bf16 if out_bf16 else f32
    planes = pl.pallas_call(
        body,
        grid=(g,),
        in_specs=([pl.BlockSpec((1, minr, k), lambda i: (i, 0, 0)),
                   pl.BlockSpec((1, nout), lambda i: (0, 0))]
                  + [wspec(w_.shape[0], w_.shape[1]) for w_ in warrs]
                  + [wspec(w_.shape[0], w_.shape[1]) for w_ in w2arrs]),
        out_specs=[pl.BlockSpec((1, min2, nout), lambda i: (i, 0, 0))] * 4,
        out_shape=[jax.ShapeDtypeStruct((g, min2, nout), odt)] * 4,
        compiler_params=pltpu.CompilerParams(dimension_semantics=("parallel",)),
    )(x_fmt, b[None, :], *[w_[None] for w_ in warrs],
      *[w_[None] for w_ in w2arrs])
    return _merge_planes(planes, h, w_sp, wp)


def _wc3wo_body(groups, ch, nchunks, wp, n3, nout, strip_h, h_img, w_img,
                x_ref, b3_ref, bo_ref, *args):
    w3_refs = args[:3]
    wo_refs = args[3:6]
    out_ref = args[6]
    h3_scr = args[7]
    m0 = wp + 16
    exn = ch + 2 * wp + 24
    q = ch // wp

    def c1(c, carry):
        base = pl.multiple_of(c * ch, 8)
        ext = x_ref[0, 0, pl.ds(base, exn), :]
        acc = None
        for gi, (off0, arels, span) in enumerate(groups):
            eh = ext[off0:off0 + span * wp + ch]
            lh = jnp.concatenate([eh[a * wp:a * wp + ch] for a in arels], axis=1)
            d = jnp.dot(lh, w3_refs[gi][0], preferred_element_type=f32)
            acc = d if acc is None else acc + d
        acc = acc[:, :n3] + acc[:, n3:]
        acc = jnp.maximum(acc + b3_ref[...], 0.0)
        a3 = acc.reshape(q, wp, n3)
        ti = jax.lax.broadcasted_iota(jnp.int32, (q, wp, 1), 0) + c * q
        ji = jax.lax.broadcasted_iota(jnp.int32, (q, wp, 1), 1)
        ti = ti + pl.program_id(1) * strip_h - 1
        ok = ((ti >= 1) & (ti <= h_img) & (ji >= 1) & (ji <= w_img))
        a3 = jnp.where(ok, a3, 0.0)
        h3_scr[pl.ds(m0 + base, ch), :] = a3.reshape(ch, n3).astype(bf16)
        return carry

    jax.lax.fori_loop(0, nchunks, c1, 0)

    def c2(c, carry):
        base = pl.multiple_of(c * ch, 8)
        ext = h3_scr[pl.ds(base, exn), :]
        acc = None
        for gi, (off0, arels, span) in enumerate(groups):
            eh = ext[off0:off0 + span * wp + ch]
            lh = jnp.concatenate([eh[a * wp:a * wp + ch] for a in arels], axis=1)
            d = jnp.dot(lh, wo_refs[gi][0], preferred_element_type=f32)
            acc = d if acc is None else acc + d
        acc = acc[:, :nout] + acc[:, nout:]
        acc = jnp.tanh(acc + bo_ref[...])
        out_ref[0, 0, pl.ds(m0 + base, ch), :] = acc
        return carry

    jax.lax.fori_loop(0, nchunks, c2, 0)


def _wc3wo_fused(st, w3, b3, wo, bo, wp, mout, ch, strip_h, h_img, w_img):
    wgs3 = _mk_conv_wgs(w3, wp, 'x2')
    wgso = _mk_conv_wgs(wo, wp, 'x2')
    groups = tuple((off0, tuple(arels), max(arels)) for off0, arels, _ in wgs3)
    w3a = [w for _, _, w in wgs3]
    woa = [w for _, _, w in wgso]
    n3 = b3.shape[-1]
    nout = bo.shape[-1]
    m0 = wp + 16
    min2 = mout + 2 * m0
    bsz, ns, minr, k = st.shape
    body = functools.partial(_wc3wo_body, groups, ch, mout // ch, wp, n3, nout,
                             strip_h, h_img, w_img)
    wspec = lambda kk, n: pl.BlockSpec((1, kk, n), lambda i, j: (0, 0, 0))
    out = pl.pallas_call(
        body,
        grid=(bsz, ns),
        in_specs=([pl.BlockSpec((1, 1, minr, k), lambda i, j: (i, j, 0, 0)),
                   pl.BlockSpec((1, n3), lambda i, j: (0, 0)),
                   pl.BlockSpec((1, nout), lambda i, j: (0, 0))]
                  + [wspec(w.shape[0], w.shape[1]) for w in w3a]
                  + [wspec(w.shape[0], w.shape[1]) for w in woa]),
        out_specs=pl.BlockSpec((1, 1, min2, nout), lambda i, j: (i, j, 0, 0)),
        out_shape=jax.ShapeDtypeStruct((bsz, ns, min2, nout), f32),
        scratch_shapes=[pltpu.VMEM((min2, n3), bf16)],
        compiler_params=pltpu.CompilerParams(
            dimension_semantics=("parallel", "parallel")),
    )(st, b3[None, :], bo[None, :], *[w[None] for w in w3a],
      *[w[None] for w in woa])
    return out


def kernel(x, We0, be0, We1, be1, Wi, bi, Wd1, bd1, Wc1, bc1, Wd2, bd2,
           Wc2, bc2, Wd3, bd3, Wc3, bc3, Wo, bo):
    bsz = x.shape[0]
    lat_fine = _conv_xla(x, We0, be0, stride=8)
    lat_coarse = _conv_xla(x, We1, be1, stride=16)
    gray = 0.299 * x[:, 0] + 0.587 * x[:, 1] + 0.114 * x[:, 2]
    ent = _entropy_map(gray)
    thr = jnp.quantile(ent.reshape(-1), 0.5)
    grain = (ent > thr).astype(x.dtype)
    coarse_up = jnp.repeat(jnp.repeat(lat_coarse, 2, axis=2), 2, axis=3)
    g = jnp.repeat(jnp.repeat(grain, 2, axis=1), 2, axis=2)[:, None]
    routed = g * lat_fine + (1.0 - g) * coarse_up

    r = routed.transpose(0, 2, 3, 1)
    r = jnp.pad(r, ((0, 0), (0, 0), (0, 0), (0, 4)))
    r_fmt, wp1 = _flat_format(r, 32, 32)
    wi8 = jnp.pad(Wi, ((0, 0), (0, 4), (0, 0), (0, 0)))
    h_fmt = _stage2(r_fmt, _mk_conv_wgs(wi8, wp1, 'x3two'), bi[None, :], wp1,
                    34 * 40, ch=680, act='none', npack=False, mask=(32, 32))
    d1 = _deconv_level(h_fmt, Wd1, bd1, 32, 32, wp1, ch=680, mode='x3two')
    d1_fmt, wp2 = _flat_format(d1, 64, 64)
    h1_fmt = _stage2(d1_fmt, _mk_conv_wgs(Wc1, wp2, 'x3two'), bc1[None, :], wp2,
                     66 * 72, ch=792, act='relu', npack=False, mask=(64, 64))
    d2 = _deconv_level(h1_fmt, Wd2, bd2, 64, 64, wp2, ch=792, mode='x3npack',
                       out_bf16=True)
    d2_fmt, wp3 = _flat_format(d2, 128, 128)
    h2_fmt = _stage2(d2_fmt, _mk_conv_wgs(Wc2, wp3, 'x2'), bc2[None, :], wp3,
                     130 * 136, ch=1768, act='relu', npack=True,
                     mask=(128, 128), out_bf16=True)
    d3 = _deconv_level(h2_fmt, Wd3, bd3, 128, 128, wp3, ch=1768, mode='x2',
                       out_bf16=True)
    wp4 = 264
    d3p = jnp.pad(d3, ((0, 0), (2, 2), (1, wp4 - 256 - 1), (0, 0)))
    st = jnp.stack([d3p[:, 64 * s:64 * s + 68] for s in range(4)], 1)
    st = st.reshape(bsz, 4, 68 * wp4, 128)
    m04 = wp4 + 16
    st = jnp.pad(st, ((0, 0), (0, 0), (m04, m04), (0, 0)))
    wo8 = jnp.pad(Wo, ((0, 5), (0, 0), (0, 0), (0, 0)))
    bo8 = jnp.pad(bo, ((0, 5),))
    rec_fmt = _wc3wo_fused(st, Wc3, bc3, wo8, bo8, wp4, 68 * wp4, ch=1056,
                           strip_h=64, h_img=256, w_img=256)
    rec_v = rec_fmt[:, :, m04:m04 + 68 * wp4, :].reshape(bsz, 4, 68, wp4, 8)
    rec = rec_v[:, :, 2:66, 1:257, :3].reshape(bsz, 256, 256, 3)
    rec = rec.transpose(0, 3, 1, 2)
    return rec, routed, grain, ent
